# Initial kernel scaffold; baseline (speedup 1.0000x reference)
import jax
import jax.numpy as jnp
from jax.experimental import pallas as pl


def kernel(logits):
    raise NotImplementedError("write your pallas kernel here")


# speedup vs baseline: 60.4844x; 60.4844x over previous
import jax
import jax.numpy as jnp
from jax.experimental import pallas as pl

_TOP_P = 0.95
_TOP_K = 64
_ROWS_PER_BLOCK = 8


def _sample_block(x_ref, conf_ref, idx_ref):
    x = x_ref[...]
    rows, _ = x.shape
    neg_min = jnp.finfo(x.dtype).min

    m = jnp.max(x, axis=-1, keepdims=True)
    z = jnp.sum(jnp.exp(x - m), axis=-1, keepdims=True)

    col = jax.lax.broadcasted_iota(jnp.int32, x.shape, 1)
    big = jnp.iinfo(jnp.int32).max
    x0 = jnp.min(jnp.where(x == m, col, big), axis=-1)

    vals = []
    cnts = []
    v = m
    for j in range(_TOP_K):
        if j > 0:
            v = jnp.max(jnp.where(x < v, x, neg_min), axis=-1, keepdims=True)
        vals.append(v)
        cnts.append(
            jnp.sum((x == v).astype(jnp.float32), axis=-1, keepdims=True)
        )
    vv = jnp.concatenate(vals, axis=1)
    cc = jnp.concatenate(cnts, axis=1)

    e = jnp.exp(vv - m)
    ce = cc * e

    r64 = jax.lax.broadcasted_iota(jnp.int32, (_TOP_K, _TOP_K), 0)
    c64 = jax.lax.broadcasted_iota(jnp.int32, (_TOP_K, _TOP_K), 1)
    upper = (r64 <= c64).astype(jnp.float32)
    nn = jnp.dot(cc, upper, preferred_element_type=jnp.float32)
    ee = jnp.dot(ce, upper, preferred_element_type=jnp.float32)
    n_prev = nn - cc
    e_prev = ee - ce

    in_topk = n_prev < jnp.float32(_TOP_K)
    room = jnp.float32(_TOP_P) * z - e_prev
    keep_p = jnp.floor(1.0 + room / jnp.where(e > 0, e, jnp.float32(1.0)))
    keep_p = jnp.where(e > 0, keep_p, jnp.where(room >= 0, cc, 0.0))
    kept = jnp.clip(keep_p, 0.0, cc) * in_topk.astype(jnp.float32)

    s = jnp.sum(kept * e, axis=-1)
    conf = 1.0 / s

    conf_ref[...] = jnp.broadcast_to(conf[:, None], (rows, 128))
    idx_ref[...] = jnp.broadcast_to(x0[:, None], (rows, 128))


@jax.jit
def kernel(logits):
    n, v = logits.shape
    nblk = n // _ROWS_PER_BLOCK
    conf, idx = pl.pallas_call(
        _sample_block,
        grid=(nblk,),
        in_specs=[pl.BlockSpec((_ROWS_PER_BLOCK, v), lambda i: (i, 0))],
        out_specs=[
            pl.BlockSpec((_ROWS_PER_BLOCK, 128), lambda i: (i, 0)),
            pl.BlockSpec((_ROWS_PER_BLOCK, 128), lambda i: (i, 0)),
        ],
        out_shape=[
            jax.ShapeDtypeStruct((n, 128), jnp.float32),
            jax.ShapeDtypeStruct((n, 128), jnp.int32),
        ],
    )(logits)
    return conf[:, 0], idx[:, 0]


# speedup vs baseline: 108.8766x; 1.8001x over previous
import jax
import jax.numpy as jnp
from jax.experimental import pallas as pl

_TOP_P = 0.95
_TOP_K = 64
_ROWS_PER_BLOCK = 8


def _slow_conf(x, m, z, neg_min):
    vals = []
    cnts = []
    v = m
    for j in range(_TOP_K):
        if j > 0:
            v = jnp.max(jnp.where(x < v, x, neg_min), axis=-1, keepdims=True)
        vals.append(v)
        cnts.append(
            jnp.sum((x == v).astype(jnp.float32), axis=-1, keepdims=True)
        )
    vv = jnp.concatenate(vals, axis=1)
    cc = jnp.concatenate(cnts, axis=1)

    e = jnp.exp(vv - m)
    ce = cc * e
    r64 = jax.lax.broadcasted_iota(jnp.int32, (_TOP_K, _TOP_K), 0)
    c64 = jax.lax.broadcasted_iota(jnp.int32, (_TOP_K, _TOP_K), 1)
    upper = (r64 <= c64).astype(jnp.float32)
    nn = jnp.dot(cc, upper, preferred_element_type=jnp.float32)
    ee = jnp.dot(ce, upper, preferred_element_type=jnp.float32)
    n_prev = nn - cc
    e_prev = ee - ce

    in_topk = n_prev < jnp.float32(_TOP_K)
    room = jnp.float32(_TOP_P) * z - e_prev
    keep_p = jnp.floor(1.0 + room / jnp.where(e > 0, e, jnp.float32(1.0)))
    keep_p = jnp.where(e > 0, keep_p, jnp.where(room >= 0, cc, 0.0))
    kept = jnp.clip(keep_p, 0.0, cc) * in_topk.astype(jnp.float32)
    return 1.0 / jnp.sum(kept * e, axis=-1)


def _sample_block(x_ref, conf_ref, idx_ref):
    x = x_ref[...]
    rows, _ = x.shape
    neg_min = jnp.finfo(x.dtype).min

    m = jnp.max(x, axis=-1, keepdims=True)
    exf = jnp.exp(x - m)
    z = jnp.sum(exf, axis=-1, keepdims=True)

    col = jax.lax.broadcasted_iota(jnp.int32, x.shape, 1)
    ibig = jnp.iinfo(jnp.int32).max
    x0 = jnp.min(jnp.where(x == m, col, ibig), axis=-1)

    s = jax.lax.bitcast_convert_type(x, jnp.int32)
    k = s ^ ((s >> 31) & jnp.int32(0x7FFFFFFF))

    cnt0 = jnp.sum((k >= 0).astype(jnp.float32), axis=-1, keepdims=True)
    u = jnp.where(
        cnt0 >= _TOP_K, jnp.int32(0), jnp.int32(-2147483648)
    ) * jnp.ones_like(m, dtype=jnp.int32)
    for b in range(30, -1, -1):
        ut = u | jnp.int32(1 << b)
        cnt = jnp.sum((k >= ut).astype(jnp.float32), axis=-1, keepdims=True)
        u = jnp.where(cnt >= _TOP_K, ut, u)

    ge = k >= u
    fbig = jnp.finfo(jnp.float32).max
    thresh = jnp.min(jnp.where(ge, x, fbig), axis=-1, keepdims=True)
    exp_t = jnp.exp(thresh - m)
    s_ge = jnp.sum(jnp.where(ge, exf, 0.0), axis=-1, keepdims=True)

    bind = (s_ge - exp_t) > jnp.float32(0.94) * z
    conf_fast = 1.0 / s_ge[:, 0]

    conf_ref[...] = jnp.broadcast_to(conf_fast[:, None], (rows, 128))
    idx_ref[...] = jnp.broadcast_to(x0[:, None], (rows, 128))

    @pl.when(jnp.any(bind))
    def _():
        conf_slow = _slow_conf(x, m, z, neg_min)
        conf = jnp.where(bind[:, 0], conf_slow, conf_fast)
        conf_ref[...] = jnp.broadcast_to(conf[:, None], (rows, 128))


@jax.jit
def kernel(logits):
    n, v = logits.shape
    nblk = n // _ROWS_PER_BLOCK
    conf, idx = pl.pallas_call(
        _sample_block,
        grid=(nblk,),
        in_specs=[pl.BlockSpec((_ROWS_PER_BLOCK, v), lambda i: (i, 0))],
        out_specs=[
            pl.BlockSpec((_ROWS_PER_BLOCK, 128), lambda i: (i, 0)),
            pl.BlockSpec((_ROWS_PER_BLOCK, 128), lambda i: (i, 0)),
        ],
        out_shape=[
            jax.ShapeDtypeStruct((n, 128), jnp.float32),
            jax.ShapeDtypeStruct((n, 128), jnp.int32),
        ],
    )(logits)
    return conf[:, 0], idx[:, 0]
